# baseline (speedup 1.0000x reference)
import functools

import jax
import jax.numpy as jnp
from jax import lax
from jax.experimental import pallas as pl
from jax.experimental.pallas import tpu as pltpu
from jax.experimental.pallas import tpu_sc as plsc

N = 100000
D = 128
G = 1024
CHUNK = 128
NBUF = 7
CHUNKS_BIG = 25
CHUNKS_SMALL = 24
BIG_TILES = 13
TAIL_START = BIG_TILES * CHUNKS_BIG * CHUNK + (32 - BIG_TILES) * CHUNKS_SMALL * CHUNK
TAIL = N - TAIL_START

_mesh = plsc.VectorSubcoreMesh(core_axis_name="c", subcore_axis_name="s")


@functools.partial(
    pl.kernel,
    mesh=_mesh,
    out_type=jax.ShapeDtypeStruct((2 * G, D), jnp.float32),
    scratch_types=[
        pltpu.VMEM((NBUF, CHUNK, D), jnp.float32),
        *[pltpu.VMEM((CHUNK,), jnp.int32) for _ in range(NBUF)],
        pltpu.VMEM((TAIL, D), jnp.float32),
        pltpu.VMEM((TAIL,), jnp.int32),
        pltpu.VMEM((8, D), jnp.float32),
        pltpu.VMEM_SHARED((G, D), jnp.float32),
        *[pltpu.SemaphoreType.DMA for _ in range(2 * NBUF)],
    ],
)
def _seg_pool(h_hbm, ids_hbm, out_hbm, buf, *rest):
    idbufs = rest[0:NBUF]
    tailbuf = rest[NBUF]
    tidbuf = rest[NBUF + 1]
    zbuf = rest[NBUF + 2]
    acc = rest[NBUF + 3]
    semL = rest[NBUF + 4:NBUF + 4 + NBUF]
    semS = rest[NBUF + 4 + NBUF:NBUF + 4 + 2 * NBUF]

    c = lax.axis_index("c")
    s = lax.axis_index("s")
    wid = c * 16 + s

    start = jnp.where(wid < BIG_TILES, wid * (CHUNKS_BIG * CHUNK),
                      BIG_TILES * CHUNKS_BIG * CHUNK
                      + (wid - BIG_TILES) * (CHUNKS_SMALL * CHUNK))
    nchunks = jnp.where(wid < BIG_TILES, CHUNKS_BIG, CHUNKS_SMALL)

    def _start_loads(k, b):
        off = start + k * CHUNK
        pltpu.async_copy(ids_hbm.at[pl.ds(off, CHUNK)], idbufs[b], semL[b])
        pltpu.async_copy(h_hbm.at[pl.ds(off, CHUNK)], buf.at[b], semL[b])

    def _wait_loads(k, b):
        off = start + k * CHUNK
        pltpu.make_async_copy(ids_hbm.at[pl.ds(off, CHUNK)], idbufs[b],
                              semL[b]).wait()
        pltpu.make_async_copy(h_hbm.at[pl.ds(off, CHUNK)], buf.at[b],
                              semL[b]).wait()

    def _drain_scatters(b):
        pltpu.make_async_copy(buf.at[b], acc.at[pl.ds(0, CHUNK)],
                              semS[b]).wait()

    for k in range(NBUF - 2):
        _start_loads(k, k)

    def _zrow(r, carry):
        for j in range(D // 16):
            zbuf[r, pl.ds(j * 16, 16)] = jnp.zeros((16,), jnp.float32)
        return carry

    lax.fori_loop(0, 8, _zrow, 0)
    for q in range(8):
        pltpu.sync_copy(zbuf, acc.at[pl.ds(s * 64 + q * 8, 8)])
    plsc.subcore_barrier()

    @pl.when(wid == 31)
    def _():
        pltpu.sync_copy(ids_hbm.at[pl.ds(TAIL_START, TAIL)], tidbuf)
        pltpu.sync_copy(h_hbm.at[pl.ds(TAIL_START, TAIL)], tailbuf)
        for j in range(TAIL // 16):
            idx = tidbuf[pl.ds(j * 16, 16)]
            pltpu.sync_copy(tailbuf.at[pl.ds(j * 16, 16)], acc.at[idx],
                            add=True)

    NSTEPS = -(-CHUNKS_BIG // NBUF)

    def _step(i, carry):
        for b in range(NBUF):
            k = NBUF * i + b
            prev = (b - 2) % NBUF

            @pl.when(k < nchunks)
            def _():
                @pl.when(jnp.logical_and(k >= 2, k + NBUF - 2 < nchunks))
                def _():
                    _drain_scatters(prev)

                @pl.when(k + NBUF - 2 < nchunks)
                def _():
                    _start_loads(k + NBUF - 2, prev)

                _wait_loads(k, b)
                pltpu.async_copy(buf.at[b], acc.at[idbufs[b]], semS[b],
                                 add=True)
        return carry

    lax.fori_loop(0, NSTEPS, _step, 0)

    for b in range(NBUF):
        _drain_scatters(b)

    plsc.subcore_barrier()
    pltpu.sync_copy(acc.at[pl.ds(s * 64, 64)],
                    out_hbm.at[pl.ds(c * G + s * 64, 64)])


def _mlp_body(p_ref, w1_ref, b1_ref, w2_ref, b2_ref, o_ref):
    p = p_ref[0:G, :] + p_ref[G:2 * G, :]
    hid = jnp.tanh(
        jnp.dot(p, w1_ref[...], preferred_element_type=jnp.float32)
        + b1_ref[...])
    o_ref[...] = (
        jnp.dot(hid, w2_ref[...], preferred_element_type=jnp.float32)
        + b2_ref[...])


def kernel(h, graph_ids, W1, b1, W2, b2):
    ids32 = graph_ids.astype(jnp.int32)
    partials = _seg_pool(h, ids32)
    y = pl.pallas_call(
        _mlp_body,
        out_shape=jax.ShapeDtypeStruct((G, 1), jnp.float32),
    )(partials, W1, b1.reshape(1, D), W2, b2.reshape(1, 1))
    return y


# speedup vs baseline: 1.0377x; 1.0377x over previous
import functools

import jax
import jax.numpy as jnp
from jax import lax
from jax.experimental import pallas as pl
from jax.experimental.pallas import tpu as pltpu
from jax.experimental.pallas import tpu_sc as plsc

N = 100000
D = 128
G = 1024
CHUNK = 160
NSTREAM = CHUNK // 16
NBUF = 5
CHUNKS_BIG = 20
CHUNKS_SMALL = 19
BIG_TILES = 17

_mesh = plsc.VectorSubcoreMesh(core_axis_name="c", subcore_axis_name="s")


@functools.partial(
    pl.kernel,
    mesh=_mesh,
    out_type=jax.ShapeDtypeStruct((2 * G, D), jnp.float32),
    scratch_types=[
        pltpu.VMEM((NBUF, CHUNK, D), jnp.float32),
        *[pltpu.VMEM((CHUNK,), jnp.int32) for _ in range(NBUF)],
        pltpu.VMEM((64, D), jnp.float32),
        pltpu.VMEM_SHARED((G, D), jnp.float32),
        *[pltpu.SemaphoreType.DMA for _ in range(2 * NBUF)],
    ],
)
def _seg_pool(h_hbm, ids_hbm, out_hbm, buf, *rest):
    idbufs = rest[0:NBUF]
    zbuf = rest[NBUF]
    acc = rest[NBUF + 1]
    semL = rest[NBUF + 2:NBUF + 2 + NBUF]
    semS = rest[NBUF + 2 + NBUF:NBUF + 2 + 2 * NBUF]

    c = lax.axis_index("c")
    s = lax.axis_index("s")
    wid = c * 16 + s

    start = jnp.where(wid < BIG_TILES, wid * (CHUNKS_BIG * CHUNK),
                      BIG_TILES * CHUNKS_BIG * CHUNK
                      + (wid - BIG_TILES) * (CHUNKS_SMALL * CHUNK))
    nchunks = jnp.where(wid < BIG_TILES, CHUNKS_BIG, CHUNKS_SMALL)

    def _start_loads(k, b):
        off = start + k * CHUNK
        pltpu.async_copy(ids_hbm.at[pl.ds(off, CHUNK)], idbufs[b], semL[b])
        pltpu.async_copy(h_hbm.at[pl.ds(off, CHUNK)], buf.at[b], semL[b])

    def _wait_loads(k, b):
        off = start + k * CHUNK
        pltpu.make_async_copy(ids_hbm.at[pl.ds(off, CHUNK)], idbufs[b],
                              semL[b]).wait()
        pltpu.make_async_copy(h_hbm.at[pl.ds(off, CHUNK)], buf.at[b],
                              semL[b]).wait()

    def _drain_scatters(b):
        pltpu.make_async_copy(buf.at[b], acc.at[pl.ds(0, CHUNK)],
                              semS[b]).wait()

    for k in range(NBUF - 3):
        _start_loads(k, k)

    def _zrow(r, carry):
        for j in range(D // 16):
            zbuf[r, pl.ds(j * 16, 16)] = jnp.zeros((16,), jnp.float32)
        return carry

    lax.fori_loop(0, 64, _zrow, 0)
    pltpu.sync_copy(zbuf, acc.at[pl.ds(s * 64, 64)])
    plsc.subcore_barrier()

    NSTEPS = -(-CHUNKS_BIG // NBUF)

    def _step(i, carry):
        for b in range(NBUF):
            k = NBUF * i + b
            prev = (b - 3) % NBUF

            @pl.when(k < nchunks)
            def _():
                @pl.when(jnp.logical_and(k >= 3, k + NBUF - 3 < nchunks))
                def _():
                    _drain_scatters(prev)

                @pl.when(k + NBUF - 3 < nchunks)
                def _():
                    _start_loads(k + NBUF - 3, prev)

                _wait_loads(k, b)
                for j in range(NSTREAM):
                    idx = idbufs[b][pl.ds(j * 16, 16)]
                    pltpu.async_copy(buf.at[b, pl.ds(j * 16, 16)],
                                     acc.at[idx], semS[b], add=True)
        return carry

    lax.fori_loop(0, NSTEPS, _step, 0)

    for b in range(NBUF):
        _drain_scatters(b)

    plsc.subcore_barrier()
    pltpu.sync_copy(acc.at[pl.ds(s * 64, 64)],
                    out_hbm.at[pl.ds(c * G + s * 64, 64)])


def _mlp_body(p_ref, w1_ref, b1_ref, w2_ref, b2_ref, o_ref):
    p = p_ref[0:G, :] + p_ref[G:2 * G, :]
    hid = jnp.tanh(
        jnp.dot(p, w1_ref[...], preferred_element_type=jnp.float32)
        + b1_ref[...])
    o_ref[...] = (
        jnp.dot(hid, w2_ref[...], preferred_element_type=jnp.float32)
        + b2_ref[...])


def kernel(h, graph_ids, W1, b1, W2, b2):
    ids32 = graph_ids.astype(jnp.int32)
    partials = _seg_pool(h, ids32)
    y = pl.pallas_call(
        _mlp_body,
        out_shape=jax.ShapeDtypeStruct((G, 1), jnp.float32),
    )(partials, W1, b1.reshape(1, D), W2, b2.reshape(1, 1))
    return y


# speedup vs baseline: 1.0388x; 1.0011x over previous
import functools

import jax
import jax.numpy as jnp
from jax import lax
from jax.experimental import pallas as pl
from jax.experimental.pallas import tpu as pltpu
from jax.experimental.pallas import tpu_sc as plsc

N = 100000
D = 128
G = 1024
CHUNK = 160
NSTREAM = CHUNK // 16
NBUF = 5
CHUNKS_BIG = 20
CHUNKS_SMALL = 19
BIG_TILES = 17

_mesh = plsc.VectorSubcoreMesh(core_axis_name="c", subcore_axis_name="s")


@functools.partial(
    pl.kernel,
    mesh=_mesh,
    out_type=jax.ShapeDtypeStruct((2 * G, D), jnp.float32),
    scratch_types=[
        pltpu.VMEM((NBUF, CHUNK, D), jnp.float32),
        pltpu.VMEM((CHUNKS_BIG * CHUNK,), jnp.int32),
        pltpu.VMEM((64, D), jnp.float32),
        pltpu.VMEM_SHARED((G, D), jnp.float32),
        *[pltpu.SemaphoreType.DMA for _ in range(2 * NBUF + 1)],
    ],
)
def _seg_pool(h_hbm, ids_hbm, out_hbm, buf, *rest):
    idbig = rest[0]
    zbuf = rest[1]
    acc = rest[2]
    semL = rest[3:3 + NBUF]
    semS = rest[3 + NBUF:3 + 2 * NBUF]
    semI = rest[3 + 2 * NBUF]

    c = lax.axis_index("c")
    s = lax.axis_index("s")
    wid = c * 16 + s

    start = jnp.where(wid < BIG_TILES, wid * (CHUNKS_BIG * CHUNK),
                      BIG_TILES * CHUNKS_BIG * CHUNK
                      + (wid - BIG_TILES) * (CHUNKS_SMALL * CHUNK))
    nchunks = jnp.where(wid < BIG_TILES, CHUNKS_BIG, CHUNKS_SMALL)

    def _start_loads(k, b):
        off = start + k * CHUNK
        pltpu.async_copy(h_hbm.at[pl.ds(off, CHUNK)], buf.at[b], semL[b])

    def _wait_loads(k, b):
        off = start + k * CHUNK
        pltpu.make_async_copy(h_hbm.at[pl.ds(off, CHUNK)], buf.at[b],
                              semL[b]).wait()

    def _drain_scatters(b):
        pltpu.make_async_copy(buf.at[b], acc.at[pl.ds(0, CHUNK)],
                              semS[b]).wait()

    pltpu.async_copy(ids_hbm.at[pl.ds(start, CHUNKS_SMALL * CHUNK)],
                     idbig.at[pl.ds(0, CHUNKS_SMALL * CHUNK)], semI)

    @pl.when(wid < BIG_TILES)
    def _():
        pltpu.async_copy(
            ids_hbm.at[pl.ds(start + CHUNKS_SMALL * CHUNK,
                             (CHUNKS_BIG - CHUNKS_SMALL) * CHUNK)],
            idbig.at[pl.ds(CHUNKS_SMALL * CHUNK,
                           (CHUNKS_BIG - CHUNKS_SMALL) * CHUNK)], semI)

    for k in range(NBUF - 3):
        _start_loads(k, k)

    def _zrow(r, carry):
        for j in range(D // 16):
            zbuf[r, pl.ds(j * 16, 16)] = jnp.zeros((16,), jnp.float32)
        return carry

    lax.fori_loop(0, 64, _zrow, 0)
    pltpu.sync_copy(zbuf, acc.at[pl.ds(s * 64, 64)])

    pltpu.make_async_copy(ids_hbm.at[pl.ds(start, CHUNKS_SMALL * CHUNK)],
                          idbig.at[pl.ds(0, CHUNKS_SMALL * CHUNK)],
                          semI).wait()

    @pl.when(wid < BIG_TILES)
    def _():
        pltpu.make_async_copy(
            ids_hbm.at[pl.ds(start + CHUNKS_SMALL * CHUNK,
                             (CHUNKS_BIG - CHUNKS_SMALL) * CHUNK)],
            idbig.at[pl.ds(CHUNKS_SMALL * CHUNK,
                           (CHUNKS_BIG - CHUNKS_SMALL) * CHUNK)],
            semI).wait()

    plsc.subcore_barrier()

    NSTEPS = -(-CHUNKS_BIG // NBUF)

    def _step(i, carry):
        for b in range(NBUF):
            k = NBUF * i + b
            prev = (b - 3) % NBUF

            @pl.when(k < nchunks)
            def _():
                @pl.when(jnp.logical_and(k >= 3, k + NBUF - 3 < nchunks))
                def _():
                    _drain_scatters(prev)

                @pl.when(k + NBUF - 3 < nchunks)
                def _():
                    _start_loads(k + NBUF - 3, prev)

                _wait_loads(k, b)
                for j in range(NSTREAM):
                    idx = idbig[pl.ds(k * CHUNK + j * 16, 16)]
                    pltpu.async_copy(buf.at[b, pl.ds(j * 16, 16)],
                                     acc.at[idx], semS[b], add=True)
        return carry

    lax.fori_loop(0, NSTEPS, _step, 0)

    for b in range(NBUF):
        _drain_scatters(b)

    plsc.subcore_barrier()
    pltpu.sync_copy(acc.at[pl.ds(s * 64, 64)],
                    out_hbm.at[pl.ds(c * G + s * 64, 64)])


def _mlp_body(p_ref, w1_ref, b1_ref, w2_ref, b2_ref, o_ref):
    p = p_ref[0:G, :] + p_ref[G:2 * G, :]
    hid = jnp.tanh(
        jnp.dot(p, w1_ref[...], preferred_element_type=jnp.float32)
        + b1_ref[...])
    o_ref[...] = (
        jnp.dot(hid, w2_ref[...], preferred_element_type=jnp.float32)
        + b2_ref[...])


def kernel(h, graph_ids, W1, b1, W2, b2):
    ids32 = graph_ids.astype(jnp.int32)
    partials = _seg_pool(h, ids32)
    y = pl.pallas_call(
        _mlp_body,
        out_shape=jax.ShapeDtypeStruct((G, 1), jnp.float32),
    )(partials, W1, b1.reshape(1, D), W2, b2.reshape(1, 1))
    return y
